# Initial kernel scaffold; baseline (speedup 1.0000x reference)
import jax
import jax.numpy as jnp
from jax.experimental import pallas as pl


def kernel(pos, batch, eps_raw, t):
    raise NotImplementedError("write your pallas kernel here")


# speedup vs baseline: 4.3946x; 4.3946x over previous
import functools

import numpy as np
import jax
import jax.numpy as jnp
from jax import lax
from jax.experimental import pallas as pl
from jax.experimental.pallas import tpu as pltpu

T = 1000
EMB = 128
BITS = 8
TPAD = 1024

_BETAS = np.linspace(1e-4, 0.02, T, dtype=np.float32)
_AC = np.cumprod((1.0 - _BETAS).astype(np.float32)).astype(np.float32)
_AC_PAD = np.concatenate([_AC, np.zeros(TPAD - T, np.float32)]).reshape(1, TPAD)

_LN1E4 = float(np.log(10000.0))


def _seg_kernel(bf_ref, pos_ref, eps_ref, sums_ref, *, blk, g):
    i = pl.program_id(0)

    @pl.when(i == 0)
    def _():
        sums_ref[...] = jnp.zeros_like(sums_ref)

    bf = bf_ref[...]
    gids = lax.broadcasted_iota(jnp.int32, (1, g), 1).astype(jnp.float32)
    onehot = (bf == gids).astype(jnp.float32)
    ones = jnp.ones((blk, 1), jnp.float32)
    zeros = jnp.zeros((blk, 1), jnp.float32)
    vals = jnp.concatenate([pos_ref[...], eps_ref[...], ones, zeros], axis=1)
    sums_ref[...] += lax.dot_general(
        onehot, vals, (((0,), (0,)), ((), ())),
        preferred_element_type=jnp.float32)


def _graph_kernel(sums_ref, tf_ref, ac_ref, aux_ref, cond_ref, alpha_ref,
                  sigma_ref, bits_ref, *, g):
    sums = sums_ref[...]
    counts = sums[:, 6:7]
    denom = jnp.maximum(counts, 1.0)
    mean = sums[:, 0:6] / denom

    tf = tf_ref[...]
    tiota = lax.broadcasted_iota(jnp.int32, (1, TPAD), 1).astype(jnp.float32)
    oh_t = (tf == tiota).astype(jnp.float32)
    ac_t = lax.dot_general(
        oh_t, ac_ref[...], (((1,), (1,)), ((), ())),
        preferred_element_type=jnp.float32)
    alpha = jnp.sqrt(ac_t)
    sigma = jnp.sqrt(1.0 - ac_t)
    alpha_ref[...] = alpha
    sigma_ref[...] = sigma

    half = EMB // 2
    fio = lax.broadcasted_iota(jnp.int32, (1, half), 1).astype(jnp.float32)
    freqs = jnp.exp(fio * (-_LN1E4 / half))
    args = tf * freqs
    cond_ref[...] = jnp.concatenate([jnp.sin(args), jnp.cos(args)], axis=1)

    ci = counts.astype(jnp.int32)
    bio = lax.broadcasted_iota(jnp.int32, (1, BITS), 1)
    bits_ref[...] = ((ci >> bio) & 1).astype(jnp.float32)

    pad = jnp.zeros((g, 8), jnp.float32)
    aux_ref[...] = jnp.concatenate([mean, alpha, sigma, pad], axis=1)


def _atom_kernel(bf_ref, pos_ref, eps_ref, aux_ref, condt_ref, noisy_ref,
                 eps_out_ref, cond_ref, *, g):
    bf = bf_ref[...]
    gids = lax.broadcasted_iota(jnp.int32, (1, g), 1).astype(jnp.float32)
    onehot = (bf == gids).astype(jnp.float32)
    a = jnp.dot(onehot, aux_ref[...], preferred_element_type=jnp.float32)
    cond_ref[...] = jnp.dot(onehot, condt_ref[...],
                            preferred_element_type=jnp.float32)
    mp = a[:, 0:3]
    me = a[:, 3:6]
    al = a[:, 6:7]
    sg = a[:, 7:8]
    x = pos_ref[...] - mp
    e = eps_ref[...] - me
    eps_out_ref[...] = e
    noisy_ref[...] = al * x + sg * e


def _pick_blk(n):
    for b in (1024, 1000, 512, 500, 256, 200, 128, 104, 100, 8):
        if n % b == 0 and b % 8 == 0:
            return b
    return n


def kernel(pos, batch, eps_raw, t):
    n = pos.shape[0]
    g = t.shape[0]
    blk = _pick_blk(n)
    nblk = n // blk

    bf = batch.astype(jnp.float32).reshape(n, 1)
    tf = t.astype(jnp.float32)
    ac = jnp.asarray(_AC_PAD)

    sums = pl.pallas_call(
        functools.partial(_seg_kernel, blk=blk, g=g),
        grid=(nblk,),
        in_specs=[
            pl.BlockSpec((blk, 1), lambda i: (i, 0)),
            pl.BlockSpec((blk, 3), lambda i: (i, 0)),
            pl.BlockSpec((blk, 3), lambda i: (i, 0)),
        ],
        out_specs=pl.BlockSpec((g, 8), lambda i: (0, 0)),
        out_shape=jax.ShapeDtypeStruct((g, 8), jnp.float32),
    )(bf, pos, eps_raw)

    aux, condt, alpha, sigma, bits = pl.pallas_call(
        functools.partial(_graph_kernel, g=g),
        in_specs=[
            pl.BlockSpec((g, 8), lambda: (0, 0)),
            pl.BlockSpec((g, 1), lambda: (0, 0)),
            pl.BlockSpec((1, TPAD), lambda: (0, 0)),
        ],
        out_specs=[
            pl.BlockSpec((g, 16), lambda: (0, 0)),
            pl.BlockSpec((g, EMB), lambda: (0, 0)),
            pl.BlockSpec((g, 1), lambda: (0, 0)),
            pl.BlockSpec((g, 1), lambda: (0, 0)),
            pl.BlockSpec((g, BITS), lambda: (0, 0)),
        ],
        out_shape=[
            jax.ShapeDtypeStruct((g, 16), jnp.float32),
            jax.ShapeDtypeStruct((g, EMB), jnp.float32),
            jax.ShapeDtypeStruct((g, 1), jnp.float32),
            jax.ShapeDtypeStruct((g, 1), jnp.float32),
            jax.ShapeDtypeStruct((g, BITS), jnp.float32),
        ],
    )(sums, tf, ac)

    noisy, eps, cond = pl.pallas_call(
        functools.partial(_atom_kernel, g=g),
        grid=(nblk,),
        in_specs=[
            pl.BlockSpec((blk, 1), lambda i: (i, 0)),
            pl.BlockSpec((blk, 3), lambda i: (i, 0)),
            pl.BlockSpec((blk, 3), lambda i: (i, 0)),
            pl.BlockSpec((g, 16), lambda i: (0, 0)),
            pl.BlockSpec((g, EMB), lambda i: (0, 0)),
        ],
        out_specs=[
            pl.BlockSpec((blk, 3), lambda i: (i, 0)),
            pl.BlockSpec((blk, 3), lambda i: (i, 0)),
            pl.BlockSpec((blk, EMB), lambda i: (i, 0)),
        ],
        out_shape=[
            jax.ShapeDtypeStruct((n, 3), jnp.float32),
            jax.ShapeDtypeStruct((n, 3), jnp.float32),
            jax.ShapeDtypeStruct((n, EMB), jnp.float32),
        ],
    )(bf, pos, eps_raw, aux, condt)

    return (noisy, eps, cond, alpha, sigma, bits)


# speedup vs baseline: 5.7916x; 1.3179x over previous
import functools

import numpy as np
import jax
import jax.numpy as jnp
from jax import lax
from jax.experimental import pallas as pl
from jax.experimental.pallas import tpu as pltpu

T = 1000
EMB = 128
BITS = 8
TPAD = 1024
TBL = 144

_BETAS = np.linspace(1e-4, 0.02, T, dtype=np.float32)
_AC = np.cumprod((1.0 - _BETAS).astype(np.float32)).astype(np.float32)
_AC_PAD = np.concatenate([_AC, np.zeros(TPAD - T, np.float32)]).reshape(1, TPAD)

_LN1E4 = float(np.log(10000.0))


def _seg_kernel(bfr_ref, pos_ref, eps_ref, sums_ref, *, blk, g):
    i = pl.program_id(0)

    @pl.when(i == 0)
    def _():
        sums_ref[...] = jnp.zeros_like(sums_ref)

    bfr = bfr_ref[0]
    gids = lax.broadcasted_iota(jnp.int32, (g, 1), 0).astype(jnp.float32)
    onehot_t = (gids == bfr).astype(jnp.float32).astype(jnp.bfloat16)
    ones = jnp.ones((blk, 1), jnp.float32)
    zeros = jnp.zeros((blk, 1), jnp.float32)
    vals = jnp.concatenate([pos_ref[...], eps_ref[...], ones, zeros],
                           axis=1).astype(jnp.bfloat16)
    sums_ref[...] += lax.dot_general(
        onehot_t, vals, (((1,), (0,)), ((), ())),
        preferred_element_type=jnp.float32)


def _graph_kernel(sums_ref, tf_ref, ac_ref, tbl_ref, alpha_ref,
                  sigma_ref, bits_ref, *, g):
    sums = sums_ref[...]
    counts = sums[:, 6:7]
    denom = jnp.maximum(counts, 1.0)
    mean = sums[:, 0:6] / denom

    tf = tf_ref[...]
    tiota = lax.broadcasted_iota(jnp.int32, (1, TPAD), 1).astype(jnp.float32)
    oh_t = (tf == tiota).astype(jnp.float32)
    ac_t = lax.dot_general(
        oh_t, ac_ref[...], (((1,), (1,)), ((), ())),
        preferred_element_type=jnp.float32)
    alpha = jnp.sqrt(ac_t)
    sigma = jnp.sqrt(1.0 - ac_t)
    alpha_ref[...] = alpha
    sigma_ref[...] = sigma

    half = EMB // 2
    fio = lax.broadcasted_iota(jnp.int32, (1, half), 1).astype(jnp.float32)
    freqs = jnp.exp(fio * (-_LN1E4 / half))
    args = tf * freqs
    cond = jnp.concatenate([jnp.sin(args), jnp.cos(args)], axis=1)

    ci = counts.astype(jnp.int32)
    bio = lax.broadcasted_iota(jnp.int32, (1, BITS), 1)
    bits_ref[...] = ((ci >> bio) & 1).astype(jnp.float32)

    pad = jnp.zeros((g, TBL - EMB - 8), jnp.float32)
    tbl_ref[...] = jnp.concatenate(
        [cond, mean, alpha, sigma, pad], axis=1).astype(jnp.bfloat16)


def _atom_kernel(bf_ref, pos_ref, eps_ref, tbl_ref, noisy_ref,
                 eps_out_ref, cond_ref, *, g):
    bf = bf_ref[...]
    gids = lax.broadcasted_iota(jnp.int32, (1, g), 1).astype(jnp.float32)
    onehot = (bf == gids).astype(jnp.float32).astype(jnp.bfloat16)
    gath = jnp.dot(onehot, tbl_ref[...],
                   preferred_element_type=jnp.float32)
    cond_ref[...] = gath[:, 0:EMB]
    mp = gath[:, EMB:EMB + 3]
    me = gath[:, EMB + 3:EMB + 6]
    al = gath[:, EMB + 6:EMB + 7]
    sg = gath[:, EMB + 7:EMB + 8]
    x = pos_ref[...] - mp
    e = eps_ref[...] - me
    eps_out_ref[...] = e
    noisy_ref[...] = al * x + sg * e


def _pick_blk(n):
    for b in (2000, 1024, 1000, 512, 500, 256, 200, 128, 104, 100, 8):
        if n % b == 0 and b % 8 == 0:
            return b
    return n


def kernel(pos, batch, eps_raw, t):
    n = pos.shape[0]
    g = t.shape[0]
    blk = _pick_blk(n)
    nblk = n // blk

    bf = batch.astype(jnp.float32).reshape(n, 1)
    bfr = batch.astype(jnp.float32).reshape(nblk, 1, blk)
    tf = t.astype(jnp.float32)
    ac = jnp.asarray(_AC_PAD)

    sums = pl.pallas_call(
        functools.partial(_seg_kernel, blk=blk, g=g),
        grid=(nblk,),
        in_specs=[
            pl.BlockSpec((1, 1, blk), lambda i: (i, 0, 0)),
            pl.BlockSpec((blk, 3), lambda i: (i, 0)),
            pl.BlockSpec((blk, 3), lambda i: (i, 0)),
        ],
        out_specs=pl.BlockSpec((g, 8), lambda i: (0, 0)),
        out_shape=jax.ShapeDtypeStruct((g, 8), jnp.float32),
    )(bfr, pos, eps_raw)

    tbl, alpha, sigma, bits = pl.pallas_call(
        functools.partial(_graph_kernel, g=g),
        in_specs=[
            pl.BlockSpec((g, 8), lambda: (0, 0)),
            pl.BlockSpec((g, 1), lambda: (0, 0)),
            pl.BlockSpec((1, TPAD), lambda: (0, 0)),
        ],
        out_specs=[
            pl.BlockSpec((g, TBL), lambda: (0, 0)),
            pl.BlockSpec((g, 1), lambda: (0, 0)),
            pl.BlockSpec((g, 1), lambda: (0, 0)),
            pl.BlockSpec((g, BITS), lambda: (0, 0)),
        ],
        out_shape=[
            jax.ShapeDtypeStruct((g, TBL), jnp.bfloat16),
            jax.ShapeDtypeStruct((g, 1), jnp.float32),
            jax.ShapeDtypeStruct((g, 1), jnp.float32),
            jax.ShapeDtypeStruct((g, BITS), jnp.float32),
        ],
    )(sums, tf, ac)

    noisy, eps, cond = pl.pallas_call(
        functools.partial(_atom_kernel, g=g),
        grid=(nblk,),
        in_specs=[
            pl.BlockSpec((blk, 1), lambda i: (i, 0)),
            pl.BlockSpec((blk, 3), lambda i: (i, 0)),
            pl.BlockSpec((blk, 3), lambda i: (i, 0)),
            pl.BlockSpec((g, TBL), lambda i: (0, 0)),
        ],
        out_specs=[
            pl.BlockSpec((blk, 3), lambda i: (i, 0)),
            pl.BlockSpec((blk, 3), lambda i: (i, 0)),
            pl.BlockSpec((blk, EMB), lambda i: (i, 0)),
        ],
        out_shape=[
            jax.ShapeDtypeStruct((n, 3), jnp.float32),
            jax.ShapeDtypeStruct((n, 3), jnp.float32),
            jax.ShapeDtypeStruct((n, EMB), jnp.float32),
        ],
    )(bf, pos, eps_raw, tbl)

    return (noisy, eps, cond, alpha, sigma, bits)


# speedup vs baseline: 6.0814x; 1.0500x over previous
import functools

import numpy as np
import jax
import jax.numpy as jnp
from jax import lax
from jax.experimental import pallas as pl
from jax.experimental.pallas import tpu as pltpu

T = 1000
EMB = 128
BITS = 8
TPAD = 1024
TBL = 144

_BETAS = np.linspace(1e-4, 0.02, T, dtype=np.float32)
_AC = np.cumprod((1.0 - _BETAS).astype(np.float32)).astype(np.float32)
_AC_PAD = np.concatenate([_AC, np.zeros(TPAD - T, np.float32)]).reshape(1, TPAD)

_LN1E4 = float(np.log(10000.0))


def _seg_kernel(bfr_ref, pos_ref, eps_ref, tf_ref, ac_ref,
                tbl_ref, alpha_ref, sigma_ref, bits_ref, sums_ref,
                *, blk, g, nblk):
    i = pl.program_id(0)

    @pl.when(i == 0)
    def _():
        sums_ref[...] = jnp.zeros_like(sums_ref)

    bfr = bfr_ref[0]
    gids = lax.broadcasted_iota(jnp.int32, (g, 1), 0).astype(jnp.float32)
    onehot_t = (gids == bfr).astype(jnp.float32).astype(jnp.bfloat16)
    ones = jnp.ones((blk, 1), jnp.float32)
    zeros = jnp.zeros((blk, 1), jnp.float32)
    vals = jnp.concatenate([pos_ref[...], eps_ref[...], ones, zeros],
                           axis=1).astype(jnp.bfloat16)
    sums_ref[...] += lax.dot_general(
        onehot_t, vals, (((1,), (0,)), ((), ())),
        preferred_element_type=jnp.float32)

    @pl.when(i == nblk - 1)
    def _():
        sums = sums_ref[...]
        counts = sums[:, 6:7]
        denom = jnp.maximum(counts, 1.0)
        mean = sums[:, 0:6] / denom

        tf = tf_ref[...]
        tiota = lax.broadcasted_iota(jnp.int32, (1, TPAD), 1).astype(jnp.float32)
        oh_t = (tf == tiota).astype(jnp.float32)
        ac_t = lax.dot_general(
            oh_t, ac_ref[...], (((1,), (1,)), ((), ())),
            preferred_element_type=jnp.float32)
        alpha = jnp.sqrt(ac_t)
        sigma = jnp.sqrt(1.0 - ac_t)
        alpha_ref[...] = alpha
        sigma_ref[...] = sigma

        half = EMB // 2
        fio = lax.broadcasted_iota(jnp.int32, (1, half), 1).astype(jnp.float32)
        freqs = jnp.exp(fio * (-_LN1E4 / half))
        args = tf * freqs
        cond = jnp.concatenate([jnp.sin(args), jnp.cos(args)], axis=1)

        ci = counts.astype(jnp.int32)
        bio = lax.broadcasted_iota(jnp.int32, (1, BITS), 1)
        bits_ref[...] = ((ci >> bio) & 1).astype(jnp.float32)

        pad = jnp.zeros((g, TBL - EMB - 8), jnp.float32)
        tbl_ref[...] = jnp.concatenate(
            [cond, mean, alpha, sigma, pad], axis=1).astype(jnp.bfloat16)


def _atom_kernel(bf_ref, pos_ref, eps_ref, tbl_ref, noisy_ref,
                 eps_out_ref, cond_ref, *, g):
    bf = bf_ref[...]
    gids = lax.broadcasted_iota(jnp.int32, (1, g), 1).astype(jnp.float32)
    onehot = (bf == gids).astype(jnp.float32).astype(jnp.bfloat16)
    gath = jnp.dot(onehot, tbl_ref[...],
                   preferred_element_type=jnp.float32)
    cond_ref[...] = gath[:, 0:EMB]
    mp = gath[:, EMB:EMB + 3]
    me = gath[:, EMB + 3:EMB + 6]
    al = gath[:, EMB + 6:EMB + 7]
    sg = gath[:, EMB + 7:EMB + 8]
    x = pos_ref[...] - mp
    e = eps_ref[...] - me
    eps_out_ref[...] = e
    noisy_ref[...] = al * x + sg * e


def _pick_blk(n, pref):
    for b in pref:
        if n % b == 0 and b % 8 == 0:
            return b
    return n


def kernel(pos, batch, eps_raw, t):
    n = pos.shape[0]
    g = t.shape[0]
    blk_s = _pick_blk(n, (2000, 1024, 1000, 512, 500, 256, 200, 128, 104, 8))
    nblk_s = n // blk_s
    blk_a = _pick_blk(n, (4000, 2000, 1024, 1000, 512, 500, 256, 200, 128, 104, 8))
    nblk_a = n // blk_a

    bf = batch.astype(jnp.float32).reshape(n, 1)
    bfr = batch.astype(jnp.float32).reshape(nblk_s, 1, blk_s)
    tf = t.astype(jnp.float32)
    ac = jnp.asarray(_AC_PAD)

    tbl, alpha, sigma, bits = pl.pallas_call(
        functools.partial(_seg_kernel, blk=blk_s, g=g, nblk=nblk_s),
        grid=(nblk_s,),
        in_specs=[
            pl.BlockSpec((1, 1, blk_s), lambda i: (i, 0, 0)),
            pl.BlockSpec((blk_s, 3), lambda i: (i, 0)),
            pl.BlockSpec((blk_s, 3), lambda i: (i, 0)),
            pl.BlockSpec((g, 1), lambda i: (0, 0)),
            pl.BlockSpec((1, TPAD), lambda i: (0, 0)),
        ],
        out_specs=[
            pl.BlockSpec((g, TBL), lambda i: (0, 0)),
            pl.BlockSpec((g, 1), lambda i: (0, 0)),
            pl.BlockSpec((g, 1), lambda i: (0, 0)),
            pl.BlockSpec((g, BITS), lambda i: (0, 0)),
        ],
        out_shape=[
            jax.ShapeDtypeStruct((g, TBL), jnp.bfloat16),
            jax.ShapeDtypeStruct((g, 1), jnp.float32),
            jax.ShapeDtypeStruct((g, 1), jnp.float32),
            jax.ShapeDtypeStruct((g, BITS), jnp.float32),
        ],
        scratch_shapes=[pltpu.VMEM((g, 8), jnp.float32)],
    )(bfr, pos, eps_raw, tf, ac)

    noisy, eps, cond = pl.pallas_call(
        functools.partial(_atom_kernel, g=g),
        grid=(nblk_a,),
        in_specs=[
            pl.BlockSpec((blk_a, 1), lambda i: (i, 0)),
            pl.BlockSpec((blk_a, 3), lambda i: (i, 0)),
            pl.BlockSpec((blk_a, 3), lambda i: (i, 0)),
            pl.BlockSpec((g, TBL), lambda i: (0, 0)),
        ],
        out_specs=[
            pl.BlockSpec((blk_a, 3), lambda i: (i, 0)),
            pl.BlockSpec((blk_a, 3), lambda i: (i, 0)),
            pl.BlockSpec((blk_a, EMB), lambda i: (i, 0)),
        ],
        out_shape=[
            jax.ShapeDtypeStruct((n, 3), jnp.float32),
            jax.ShapeDtypeStruct((n, 3), jnp.float32),
            jax.ShapeDtypeStruct((n, EMB), jnp.float32),
        ],
        compiler_params=pltpu.CompilerParams(
            dimension_semantics=("arbitrary",)),
    )(bf, pos, eps_raw, tbl)

    return (noisy, eps, cond, alpha, sigma, bits)
